# baseline (speedup 1.0000x reference)
import jax
import jax.numpy as jnp
from jax.experimental import pallas as pl
from jax.experimental.pallas import tpu as pltpu

E = 64
D = 2048
TILE = 512
HALVES = 2
KSTEPS = D // HALVES // TILE


def _mlp_kernel(x_ref,
                gamma_ref, beta_ref,
                w1_ref,
                b1_ref,
                w2_ref,
                out_ref,
                xln_ref,
                acc_ref):
    k = pl.program_id(1)

    @pl.when(k == 0)
    def _init():
        x = x_ref[:, 7, :]
        mu = jnp.mean(x, axis=-1, keepdims=True)
        var = jnp.mean((x - mu) ** 2, axis=-1, keepdims=True)
        xln_ref[...] = ((x - mu) * jax.lax.rsqrt(var + 1e-5)
                        * gamma_ref[...] + beta_ref[...])
        acc_ref[...] = jnp.zeros_like(acc_ref)

    h1 = jnp.dot(xln_ref[...], w1_ref[...],
                 preferred_element_type=jnp.float32) + b1_ref[...]
    h1 = 0.5 * h1 * (1.0 + jax.lax.erf(h1 * 0.7071067811865476))
    acc_ref[...] += jnp.dot(h1, w2_ref[...],
                            preferred_element_type=jnp.float32)

    @pl.when(k == KSTEPS - 1)
    def _finish():
        out_ref[0] = acc_ref[...]


def _epilogue_kernel(idx_ref,
                     part_ref,
                     b2_ref,
                     emb_ref,
                     cent_ref,
                     spat_ref,
                     comp_ref,
                     out_ref):
    h2 = part_ref[0] + part_ref[1] + b2_ref[...]
    att = jax.lax.dot_general(
        h2, emb_ref[...], (((1,), (1,)), ((), ())),
        preferred_element_type=jnp.float32) * (1.0 / (D ** 0.5))
    cent = jax.nn.softmax(cent_ref[...], axis=-1)
    idx = idx_ref[0, 0]
    row = spat_ref[pl.ds(idx, 1), :] + comp_ref[pl.ds(idx, 1), :]
    eids = jax.lax.broadcasted_iota(jnp.int32, (1, E), 1)
    row = row + jnp.where(eids == idx, -0.1, 0.0)
    out_ref[...] = jax.nn.softmax(att + cent + row, axis=-1)


def kernel(hidden_states, ln_gamma, ln_beta, W1, b1, W2, b2,
           expert_embeddings, centrality_bias, spatial_bias,
           compatibility_matrix, current_expert_idx):
    B, S, d = hidden_states.shape
    partial = pl.pallas_call(
        _mlp_kernel,
        grid=(HALVES, KSTEPS),
        in_specs=[
            pl.BlockSpec((B, 8, d), lambda c, k: (0, S // 8 - 1, 0)),
            pl.BlockSpec((1, d), lambda c, k: (0, 0)),
            pl.BlockSpec((1, d), lambda c, k: (0, 0)),
            pl.BlockSpec((d, TILE), lambda c, k: (0, c * KSTEPS + k)),
            pl.BlockSpec((1, TILE), lambda c, k: (0, c * KSTEPS + k)),
            pl.BlockSpec((TILE, d), lambda c, k: (c * KSTEPS + k, 0)),
        ],
        out_specs=pl.BlockSpec((1, B, d), lambda c, k: (c, 0, 0)),
        scratch_shapes=[
            pltpu.VMEM((B, d), jnp.float32),
            pltpu.VMEM((B, d), jnp.float32),
        ],
        out_shape=jax.ShapeDtypeStruct((HALVES, B, d), jnp.float32),
        compiler_params=pltpu.CompilerParams(
            dimension_semantics=("parallel", "arbitrary"),
        ),
    )(hidden_states,
      ln_gamma.reshape(1, d), ln_beta.reshape(1, d),
      W1, b1.reshape(1, d), W2)

    idx = jnp.asarray(current_expert_idx, jnp.int32).reshape(1, 1)
    grid_spec = pltpu.PrefetchScalarGridSpec(
        num_scalar_prefetch=1,
        grid=(1,),
        in_specs=[
            pl.BlockSpec((HALVES, B, d), lambda j, *_: (0, 0, 0)),
            pl.BlockSpec((1, d), lambda j, *_: (0, 0)),
            pl.BlockSpec((E, d), lambda j, *_: (0, 0)),
            pl.BlockSpec((1, E), lambda j, *_: (0, 0)),
            pl.BlockSpec((E, E), lambda j, *_: (0, 0)),
            pl.BlockSpec((E, E), lambda j, *_: (0, 0)),
        ],
        out_specs=pl.BlockSpec((B, E), lambda j, *_: (0, 0)),
    )
    return pl.pallas_call(
        _epilogue_kernel,
        grid_spec=grid_spec,
        out_shape=jax.ShapeDtypeStruct((B, E), jnp.float32),
    )(idx, partial, b2.reshape(1, d), expert_embeddings,
      centrality_bias.reshape(1, E), spatial_bias, compatibility_matrix)


# speedup vs baseline: 1.0217x; 1.0217x over previous
import jax
import jax.numpy as jnp
from jax.experimental import pallas as pl
from jax.experimental.pallas import tpu as pltpu

E = 64
D = 2048
TILE = 512
NT = D // TILE


def _router_kernel(idx_ref,
                   x_ref,
                   gamma_ref, beta_ref,
                   w1_ref,
                   b1_ref,
                   w2_ref,
                   b2_ref,
                   emb_ref,
                   cent_ref,
                   spat_ref,
                   comp_ref,
                   out_ref,
                   xln_ref,
                   h1_ref,
                   acc_ref):
    i = pl.program_id(0)

    @pl.when(i == 0)
    def _init():
        x = x_ref[:, 7, :]
        mu = jnp.mean(x, axis=-1, keepdims=True)
        var = jnp.mean((x - mu) ** 2, axis=-1, keepdims=True)
        xln_ref[...] = ((x - mu) * jax.lax.rsqrt(var + 1e-5)
                        * gamma_ref[...] + beta_ref[...])
        h1_ref[...] = jnp.zeros_like(h1_ref)
        acc_ref[...] = jnp.zeros_like(acc_ref)

    @pl.when(i < NT)
    def _first_gemm():
        xs = xln_ref[:, pl.ds(i * TILE, TILE)]
        h1_ref[...] += jnp.dot(xs, w1_ref[...],
                               preferred_element_type=jnp.float32)

    @pl.when(i == NT - 1)
    def _gelu():
        h1 = h1_ref[...] + b1_ref[...]
        h1_ref[...] = 0.5 * h1 * (1.0 + jax.lax.erf(h1 * 0.7071067811865476))

    @pl.when(i >= NT)
    def _second_gemm():
        hs = h1_ref[:, pl.ds((i - NT) * TILE, TILE)]
        acc_ref[...] += jnp.dot(hs, w2_ref[...],
                                preferred_element_type=jnp.float32)

    @pl.when(i == 2 * NT - 1)
    def _finish():
        h2 = acc_ref[...] + b2_ref[...]
        att = jax.lax.dot_general(
            h2, emb_ref[...], (((1,), (1,)), ((), ())),
            preferred_element_type=jnp.float32) * (1.0 / (D ** 0.5))
        cent = jax.nn.softmax(cent_ref[...], axis=-1)
        idx = idx_ref[0, 0]
        row = spat_ref[pl.ds(idx, 1), :] + comp_ref[pl.ds(idx, 1), :]
        eids = jax.lax.broadcasted_iota(jnp.int32, (1, E), 1)
        row = row + jnp.where(eids == idx, -0.1, 0.0)
        out_ref[...] = jax.nn.softmax(att + cent + row, axis=-1)


def kernel(hidden_states, ln_gamma, ln_beta, W1, b1, W2, b2,
           expert_embeddings, centrality_bias, spatial_bias,
           compatibility_matrix, current_expert_idx):
    B, S, d = hidden_states.shape
    idx = jnp.asarray(current_expert_idx, jnp.int32).reshape(1, 1)
    grid_spec = pltpu.PrefetchScalarGridSpec(
        num_scalar_prefetch=1,
        grid=(2 * NT,),
        in_specs=[
            pl.BlockSpec((B, 8, d), lambda i, *_: (0, S // 8 - 1, 0)),
            pl.BlockSpec((1, d), lambda i, *_: (0, 0)),
            pl.BlockSpec((1, d), lambda i, *_: (0, 0)),
            pl.BlockSpec((TILE, d), lambda i, *_: (jnp.minimum(i, NT - 1), 0)),
            pl.BlockSpec((1, d), lambda i, *_: (0, 0)),
            pl.BlockSpec((TILE, d), lambda i, *_: (jnp.maximum(i - NT, 0), 0)),
            pl.BlockSpec((1, d), lambda i, *_: (0, 0)),
            pl.BlockSpec((E, d), lambda i, *_: (0, 0)),
            pl.BlockSpec((1, E), lambda i, *_: (0, 0)),
            pl.BlockSpec((E, E), lambda i, *_: (0, 0)),
            pl.BlockSpec((E, E), lambda i, *_: (0, 0)),
        ],
        out_specs=pl.BlockSpec((B, E), lambda i, *_: (0, 0)),
        scratch_shapes=[
            pltpu.VMEM((B, d), jnp.float32),
            pltpu.VMEM((B, d), jnp.float32),
            pltpu.VMEM((B, d), jnp.float32),
        ],
    )
    return pl.pallas_call(
        _router_kernel,
        grid_spec=grid_spec,
        out_shape=jax.ShapeDtypeStruct((B, E), jnp.float32),
        compiler_params=pltpu.CompilerParams(
            dimension_semantics=("arbitrary",),
        ),
    )(idx,
      hidden_states,
      ln_gamma.reshape(1, d), ln_beta.reshape(1, d),
      W1, b1.reshape(1, d),
      W2, b2.reshape(1, d),
      expert_embeddings,
      centrality_bias.reshape(1, E),
      spatial_bias, compatibility_matrix)


# speedup vs baseline: 1.1367x; 1.1126x over previous
import jax
import jax.numpy as jnp
from jax.experimental import pallas as pl
from jax.experimental.pallas import tpu as pltpu

E = 64
D = 2048
TILE = 512
NSTEPS = 2


def _router_kernel(idx_ref,
                   x_ref,
                   gamma_ref, beta_ref,
                   w1a_ref, w1b_ref,
                   b1a_ref, b1b_ref,
                   w2a_ref, w2b_ref,
                   b2_ref,
                   emb_ref,
                   cent_ref,
                   spat_ref,
                   comp_ref,
                   out_ref,
                   xln_ref,
                   acc_ref):
    j = pl.program_id(0)

    @pl.when(j == 0)
    def _init():
        x = x_ref[:, 7, :]
        mu = jnp.mean(x, axis=-1, keepdims=True)
        var = jnp.mean((x - mu) ** 2, axis=-1, keepdims=True)
        xln_ref[...] = ((x - mu) * jax.lax.rsqrt(var + 1e-5)
                        * gamma_ref[...] + beta_ref[...])
        acc_ref[...] = jnp.zeros_like(acc_ref)

    xln = xln_ref[...]
    c = 0.7071067811865476
    h1a = jnp.dot(xln, w1a_ref[...],
                  preferred_element_type=jnp.float32) + b1a_ref[...]
    h1a = 0.5 * h1a * (1.0 + jax.lax.erf(h1a * c))
    h1b = jnp.dot(xln, w1b_ref[...],
                  preferred_element_type=jnp.float32) + b1b_ref[...]
    h1b = 0.5 * h1b * (1.0 + jax.lax.erf(h1b * c))
    acc_ref[...] += (jnp.dot(h1a, w2a_ref[...],
                             preferred_element_type=jnp.float32)
                     + jnp.dot(h1b, w2b_ref[...],
                               preferred_element_type=jnp.float32))

    @pl.when(j == NSTEPS - 1)
    def _finish():
        h2 = acc_ref[...] + b2_ref[...]
        att = jax.lax.dot_general(
            h2, emb_ref[...], (((1,), (1,)), ((), ())),
            preferred_element_type=jnp.float32) * (1.0 / (D ** 0.5))
        cent = jax.nn.softmax(cent_ref[...], axis=-1)
        idx = idx_ref[0, 0]
        row = spat_ref[pl.ds(idx, 1), :] + comp_ref[pl.ds(idx, 1), :]
        eids = jax.lax.broadcasted_iota(jnp.int32, (1, E), 1)
        row = row + jnp.where(eids == idx, -0.1, 0.0)
        out_ref[...] = jax.nn.softmax(att + cent + row, axis=-1)


def kernel(hidden_states, ln_gamma, ln_beta, W1, b1, W2, b2,
           expert_embeddings, centrality_bias, spatial_bias,
           compatibility_matrix, current_expert_idx):
    B, S, d = hidden_states.shape
    idx = jnp.asarray(current_expert_idx, jnp.int32).reshape(1, 1)
    grid_spec = pltpu.PrefetchScalarGridSpec(
        num_scalar_prefetch=1,
        grid=(NSTEPS,),
        in_specs=[
            pl.BlockSpec((B, 8, d), lambda j, *_: (0, S // 8 - 1, 0)),
            pl.BlockSpec((1, d), lambda j, *_: (0, 0)),
            pl.BlockSpec((1, d), lambda j, *_: (0, 0)),
            pl.BlockSpec((d, TILE), lambda j, *_: (0, j)),
            pl.BlockSpec((d, TILE), lambda j, *_: (0, j + NSTEPS)),
            pl.BlockSpec((1, TILE), lambda j, *_: (0, j)),
            pl.BlockSpec((1, TILE), lambda j, *_: (0, j + NSTEPS)),
            pl.BlockSpec((TILE, d), lambda j, *_: (j, 0)),
            pl.BlockSpec((TILE, d), lambda j, *_: (j + NSTEPS, 0)),
            pl.BlockSpec((1, d), lambda j, *_: (0, 0)),
            pl.BlockSpec((E, d), lambda j, *_: (0, 0)),
            pl.BlockSpec((1, E), lambda j, *_: (0, 0)),
            pl.BlockSpec((E, E), lambda j, *_: (0, 0)),
            pl.BlockSpec((E, E), lambda j, *_: (0, 0)),
        ],
        out_specs=pl.BlockSpec((B, E), lambda j, *_: (0, 0)),
        scratch_shapes=[
            pltpu.VMEM((B, d), jnp.float32),
            pltpu.VMEM((B, d), jnp.float32),
        ],
    )
    return pl.pallas_call(
        _router_kernel,
        grid_spec=grid_spec,
        out_shape=jax.ShapeDtypeStruct((B, E), jnp.float32),
        compiler_params=pltpu.CompilerParams(
            dimension_semantics=("arbitrary",),
        ),
    )(idx,
      hidden_states,
      ln_gamma.reshape(1, d), ln_beta.reshape(1, d),
      W1, W1, b1.reshape(1, d), b1.reshape(1, d),
      W2, W2, b2.reshape(1, d),
      expert_embeddings,
      centrality_bias.reshape(1, E),
      spatial_bias, compatibility_matrix)
